# Initial kernel scaffold; baseline (speedup 1.0000x reference)
import jax
import jax.numpy as jnp
from jax.experimental import pallas as pl


def kernel(x, edge_index, Wl0, Wr0, b0, Wl1, Wr1, b1, Wl2, Wr2, b2, Wfc, bfc):
    raise NotImplementedError("write your pallas kernel here")


# speedup vs baseline: 6.5038x; 6.5038x over previous
import functools

import jax
import jax.numpy as jnp
from jax import lax
from jax.experimental import pallas as pl
from jax.experimental.pallas import tpu as pltpu
from jax.experimental.pallas import tpu_sc as plsc

N = 10000
E = 320000
D = 128
D_OUT = 16

NC = 2
NS = 16
NW = NC * NS
EPW = E // NW
K = 100
NCH = EPW // K
NPAD = 10240
RPT = NPAD // NS

_MESH = plsc.VectorSubcoreMesh(core_axis_name="c", subcore_axis_name="s")
_PART = jax.ShapeDtypeStruct((NC, NPAD, D), jnp.float32)


@functools.partial(
    pl.kernel, mesh=_MESH, out_type=[_PART],
    scratch_types=[
        pltpu.VMEM((NCH, K), jnp.int32),
        pltpu.VMEM((NCH, K), jnp.int32),
        pltpu.VMEM((K, D), jnp.float32),
        pltpu.VMEM_SHARED((NPAD, D), jnp.float32),
        pltpu.SemaphoreType.DMA,
    ])
def _sc_agg(h_hbm, src_hbm, dst_hbm, z_hbm, agg_out,
            srcv, dstv, rows0, agg_sh, sem0):
  c = lax.axis_index("c")
  s = lax.axis_index("s")
  wid = s * NC + c
  row0 = s * RPT

  pltpu.sync_copy(src_hbm.at[wid], srcv)
  pltpu.sync_copy(dst_hbm.at[wid], dstv)
  pltpu.sync_copy(z_hbm, agg_sh.at[pl.ds(row0, RPT)])
  plsc.subcore_barrier()

  def step(i, carry):
    pltpu.async_copy(h_hbm.at[srcv.at[i]], rows0, sem0).wait()
    pltpu.sync_copy(rows0, agg_sh.at[dstv.at[i]], add=True)
    return carry

  lax.fori_loop(0, NCH, step, 0)
  plsc.subcore_barrier()

  pltpu.sync_copy(agg_sh.at[pl.ds(row0, RPT)],
                  agg_out.at[c, pl.ds(row0, RPT)])


@functools.partial(
    pl.kernel, mesh=_MESH, out_type=[_PART],
    scratch_types=[
        pltpu.VMEM((NCH, K), jnp.int32),
        pltpu.VMEM((K, D), jnp.float32),
        pltpu.VMEM_SHARED((NPAD, D), jnp.float32),
    ])
def _sc_cnt(dst_hbm, ones_hbm, z_hbm, cnt_out, dstv, onesv, cnt_sh):
  c = lax.axis_index("c")
  s = lax.axis_index("s")
  wid = s * NC + c
  row0 = s * RPT

  pltpu.sync_copy(dst_hbm.at[wid], dstv)
  pltpu.sync_copy(ones_hbm, onesv)
  pltpu.sync_copy(z_hbm, cnt_sh.at[pl.ds(row0, RPT)])
  plsc.subcore_barrier()

  def step(i, carry):
    pltpu.sync_copy(onesv, cnt_sh.at[dstv.at[i]], add=True)
    return carry

  lax.fori_loop(0, NCH, step, 0)
  plsc.subcore_barrier()
  pltpu.sync_copy(cnt_sh.at[pl.ds(row0, RPT)],
                  cnt_out.at[c, pl.ds(row0, RPT)])


BN = 1000


def _tc_mid_body(h_ref, a0_ref, a1_ref, c0_ref, c1_ref, wl_ref, wr_ref,
                 b_ref, out_ref):
  agg = a0_ref[...] + a1_ref[...]
  cnt = c0_ref[...] + c1_ref[...]
  inv = 1.0 / jnp.maximum(cnt[:, 0:1], 1.0)
  mean = agg * inv
  acc = jnp.dot(mean, wl_ref[...], preferred_element_type=jnp.float32)
  acc += jnp.dot(h_ref[...], wr_ref[...], preferred_element_type=jnp.float32)
  acc += b_ref[...]
  out_ref[...] = jnp.maximum(acc, 0.0)


def _tc_final_body(h_ref, a0_ref, a1_ref, c0_ref, c1_ref, wl_ref, wr_ref,
                   b_ref, wfc_ref, bfc_ref, emb_ref, logp_ref):
  agg = a0_ref[...] + a1_ref[...]
  cnt = c0_ref[...] + c1_ref[...]
  inv = 1.0 / jnp.maximum(cnt[:, 0:1], 1.0)
  mean = agg * inv
  emb = jnp.dot(mean, wl_ref[...], preferred_element_type=jnp.float32)
  emb += jnp.dot(h_ref[...], wr_ref[...], preferred_element_type=jnp.float32)
  emb += b_ref[...]
  emb_ref[...] = emb
  y = jnp.dot(emb, wfc_ref[...], preferred_element_type=jnp.float32)
  y += bfc_ref[...]
  m = jnp.max(y, axis=1, keepdims=True)
  lse = m + jnp.log(jnp.sum(jnp.exp(y - m), axis=1, keepdims=True))
  logp_ref[...] = y - lse


def _row_spec(width):
  return pl.BlockSpec((BN, width), lambda i: (i, 0))


def _full_spec(rows, width):
  return pl.BlockSpec((rows, width), lambda i: (0, 0))


def _tc_mid(h, a0, a1, c0, c1, wl, wr, b):
  return pl.pallas_call(
      _tc_mid_body,
      grid=(N // BN,),
      in_specs=[
          _row_spec(D), _row_spec(D), _row_spec(D),
          _row_spec(D), _row_spec(D),
          _full_spec(D, D), _full_spec(D, D), _full_spec(1, D),
      ],
      out_specs=_row_spec(D),
      out_shape=jax.ShapeDtypeStruct((N, D), jnp.float32),
  )(h, a0, a1, c0, c1, wl, wr, b)


def _tc_final(h, a0, a1, c0, c1, wl, wr, b, wfc, bfc):
  return pl.pallas_call(
      _tc_final_body,
      grid=(N // BN,),
      in_specs=[
          _row_spec(D), _row_spec(D), _row_spec(D),
          _row_spec(D), _row_spec(D),
          _full_spec(D, D), _full_spec(D, D), _full_spec(1, D),
          _full_spec(D, D_OUT), _full_spec(1, D_OUT),
      ],
      out_specs=[_row_spec(D), _row_spec(D_OUT)],
      out_shape=[
          jax.ShapeDtypeStruct((N, D), jnp.float32),
          jax.ShapeDtypeStruct((N, D_OUT), jnp.float32),
      ],
  )(h, a0, a1, c0, c1, wl, wr, b, wfc, bfc)


@jax.jit
def kernel(x, edge_index, Wl0, Wr0, b0, Wl1, Wr1, b1, Wl2, Wr2, b2, Wfc, bfc):
  src = edge_index[0].reshape(NW, NCH, K)
  dst = edge_index[1].reshape(NW, NCH, K)
  zeros = jnp.zeros((RPT, D), jnp.float32)
  ones = jnp.ones((K, D), jnp.float32)
  b0r = b0.reshape(1, D)
  b1r = b1.reshape(1, D)
  b2r = b2.reshape(1, D)
  bfcr = bfc.reshape(1, D_OUT)

  (cnt,) = _sc_cnt(dst, ones, zeros)
  c0 = cnt[0]
  c1 = cnt[1]
  (agg0,) = _sc_agg(x, src, dst, zeros)
  h1 = _tc_mid(x, agg0[0], agg0[1], c0, c1, Wl0, Wr0, b0r)
  (agg1,) = _sc_agg(h1, src, dst, zeros)
  h2 = _tc_mid(h1, agg1[0], agg1[1], c0, c1, Wl1, Wr1, b1r)
  (agg2,) = _sc_agg(h2, src, dst, zeros)
  emb, logp = _tc_final(h2, agg2[0], agg2[1], c0, c1, Wl2, Wr2, b2r,
                        Wfc, bfcr)
  return (emb, logp)
